# Initial kernel scaffold; baseline (speedup 1.0000x reference)
import functools

import jax
import jax.numpy as jnp
import numpy as np
from jax import lax
from jax.experimental import pallas as pl
from jax.experimental.pallas import tpu as pltpu
from jax.experimental.pallas import tpu_sc as plsc

N = 10000
E = 160000
D = 128
C = 280
CP = 288
CH = 144
EDP = 16
NC = 2
NS = 16
NW = NC * NS
INV_SQRT_C = 1.0 / float(np.sqrt(float(C)))

EPT1 = E // NW
B1 = 40
NB1 = EPT1 // B1
EPT2 = E // NS
B2 = 80
NB2 = EPT2 // B2

_mesh = plsc.VectorSubcoreMesh(
    core_axis_name="c", subcore_axis_name="s", num_cores=NC, num_subcores=NS)


def _projc_body(x_ref, wq, bq, wk, bk, wv, bv, wet, q_out, k_out, qe_out, v_out):
    x = x_ref[...]
    z8 = jnp.zeros((x.shape[0], CP - C), jnp.float32)
    z4 = jnp.zeros((x.shape[0], 4), jnp.float32)
    qb = jnp.dot(x, wq[...], preferred_element_type=jnp.float32) + bq[...]
    kb = jnp.dot(x, wk[...], preferred_element_type=jnp.float32) + bk[...]
    vb = jnp.dot(x, wv[...], preferred_element_type=jnp.float32) + bv[...]
    q_out[...] = jnp.concatenate([qb, z8], axis=1)
    k_out[...] = jnp.concatenate([kb, z8], axis=1)
    qe_out[...] = jnp.dot(qb, wet[...], preferred_element_type=jnp.float32)
    v_out[...] = jnp.concatenate(
        [vb[:, :140], z4, vb[:, 140:], z4], axis=1)


def _phase_a(x, Wq, bq, Wk, bk, Wv, bv, WeT16):
    bn = 1000
    grid = (N // bn,)
    full = lambda shp: pl.BlockSpec(shp, lambda i: (0, 0))
    return pl.pallas_call(
        _projc_body,
        grid=grid,
        in_specs=[
            pl.BlockSpec((bn, D), lambda i: (i, 0)),
            full((D, C)), full((1, C)),
            full((D, C)), full((1, C)),
            full((D, C)), full((1, C)),
            full((C, EDP)),
        ],
        out_specs=[
            pl.BlockSpec((bn, CP), lambda i: (i, 0)),
            pl.BlockSpec((bn, CP), lambda i: (i, 0)),
            pl.BlockSpec((bn, EDP), lambda i: (i, 0)),
            pl.BlockSpec((bn, CP), lambda i: (i, 0)),
        ],
        out_shape=[
            jax.ShapeDtypeStruct((N, CP), jnp.float32),
            jax.ShapeDtypeStruct((N, CP), jnp.float32),
            jax.ShapeDtypeStruct((N, EDP), jnp.float32),
            jax.ShapeDtypeStruct((N, CP), jnp.float32),
        ],
    )(x, Wq, bq.reshape(1, C), Wk, bk.reshape(1, C), Wv, bv.reshape(1, C),
      WeT16)


def _sc_pass1_body(q_hbm, k_hbm, qe_hbm, ei_hbm, ea_hbm,
                   p_hbm, dparts_hbm,
                   qbuf, kbuf, qebuf, eabuf, srcb, dstb, sbuf, pbuf, denomb,
                   sem):
    wid = lax.axis_index("c") * NS + lax.axis_index("s")
    z16 = jnp.zeros((16,), jnp.float32)

    def zdenom(i, _):
        denomb[pl.ds(i * 16, 16)] = z16
        return 0
    lax.fori_loop(0, N // 16, zdenom, 0)

    def block(i, _):
        base = wid * EPT1 + i * B1
        pltpu.sync_copy(ei_hbm.at[0, pl.ds(base, B1)], srcb)
        pltpu.sync_copy(ei_hbm.at[1, pl.ds(base, B1)], dstb)
        pltpu.sync_copy(ea_hbm.at[pl.ds(base, B1)], eabuf)
        c1 = pltpu.async_copy(k_hbm.at[srcb], kbuf, sem)
        c2 = pltpu.async_copy(q_hbm.at[dstb], qbuf, sem)
        c3 = pltpu.async_copy(qe_hbm.at[dstb], qebuf, sem)
        c1.wait()
        c2.wait()
        c3.wait()

        def edot(e, _):
            acc = kbuf[e, pl.ds(0, 16)] * qbuf[e, pl.ds(0, 16)]
            for j in range(1, CP // 16):
                acc = acc + kbuf[e, pl.ds(16 * j, 16)] * qbuf[e, pl.ds(16 * j, 16)]
            acc = acc + qebuf[e, :] * eabuf[e, :]
            sbuf[e] = jnp.sum(acc) * INV_SQRT_C
            return 0
        lax.fori_loop(0, B1, edot, 0)

        def evec(j, _):
            pbuf[pl.ds(16 * j, 16)] = jnp.exp(sbuf[pl.ds(16 * j, 16)])
            return 0
        lax.fori_loop(0, B1 // 16, evec, 0)

        def edenom(e, _):
            d = dstb[e]
            denomb[d] = denomb[d] + pbuf[e]
            return 0
        lax.fori_loop(0, B1, edenom, 0)

        pltpu.sync_copy(pbuf, p_hbm.at[pl.ds(base, B1)])
        return 0

    lax.fori_loop(0, NB1, block, 0)
    pltpu.sync_copy(denomb, dparts_hbm.at[wid])


_sc_pass1 = functools.partial(
    pl.kernel,
    out_type=[
        jax.ShapeDtypeStruct((E,), jnp.float32),
        jax.ShapeDtypeStruct((NW, N), jnp.float32),
    ],
    mesh=_mesh,
    scratch_types=[
        pltpu.VMEM((B1, CP), jnp.float32),
        pltpu.VMEM((B1, CP), jnp.float32),
        pltpu.VMEM((B1, EDP), jnp.float32),
        pltpu.VMEM((B1, EDP), jnp.float32),
        pltpu.VMEM((B1,), jnp.int32),
        pltpu.VMEM((B1,), jnp.int32),
        pltpu.VMEM((B1,), jnp.float32),
        pltpu.VMEM((B1,), jnp.float32),
        pltpu.VMEM((N,), jnp.float32),
        pltpu.SemaphoreType.DMA,
    ],
)(_sc_pass1_body)


def _sc_pass2_body(vab_hbm, ei_hbm, ea_hbm, p_hbm, dparts_hbm,
                   out_hbm, eacc_hbm,
                   acc_sh, eacc_sh,
                   dbuf, tmp, zbuf, z16buf, srcb, dstb, pbuf, wbuf,
                   vrows, eabuf, easc, sem):
    c = lax.axis_index("c")
    s = lax.axis_index("s")
    z16 = jnp.zeros((16,), jnp.float32)
    rows_per_tile = N // NS

    def zrow(r, _):
        for j in range(CH // 16):
            zbuf[r, pl.ds(16 * j, 16)] = z16
        return 0
    lax.fori_loop(0, 125, zrow, 0)

    def zrow16(r, _):
        z16buf[r, :] = z16
        return 0
    lax.fori_loop(0, rows_per_tile, zrow16, 0)

    for t in range(5):
        pltpu.sync_copy(zbuf, acc_sh.at[pl.ds(s * rows_per_tile + t * 125, 125)])

    @pl.when(c == 0)
    def _():
        pltpu.sync_copy(z16buf, eacc_sh.at[pl.ds(s * rows_per_tile, rows_per_tile)])

    def zd(i, _):
        dbuf[pl.ds(i * 16, 16)] = z16
        return 0
    lax.fori_loop(0, N // 16, zd, 0)

    def addpart(r, _):
        pltpu.sync_copy(dparts_hbm.at[r], tmp)

        def addv(j, _):
            dbuf[pl.ds(16 * j, 16)] = dbuf[pl.ds(16 * j, 16)] + tmp[pl.ds(16 * j, 16)]
            return 0
        lax.fori_loop(0, N // 16, addv, 0)
        return 0
    lax.fori_loop(0, NW, addpart, 0)

    plsc.subcore_barrier()

    coff = c * N

    def block(i, _):
        base = s * EPT2 + i * B2
        pltpu.sync_copy(ei_hbm.at[0, pl.ds(base, B2)], srcb)
        pltpu.sync_copy(ei_hbm.at[1, pl.ds(base, B2)], dstb.at[0])
        pltpu.sync_copy(p_hbm.at[pl.ds(base, B2)], pbuf)

        def soff(j, _):
            srcb[pl.ds(16 * j, 16)] = srcb[pl.ds(16 * j, 16)] + coff
            return 0
        lax.fori_loop(0, B2 // 16, soff, 0)

        pltpu.async_copy(vab_hbm.at[srcb], vrows, sem).wait()

        def wvec(j, _):
            d16 = plsc.load_gather(dbuf, [dstb[0, pl.ds(16 * j, 16)]])
            wbuf[pl.ds(16 * j, 16)] = pbuf[pl.ds(16 * j, 16)] / (d16 + 1e-16)
            return 0
        lax.fori_loop(0, B2 // 16, wvec, 0)

        def scale(e, _):
            w = wbuf[e]
            for j in range(CH // 16):
                vrows[e, pl.ds(16 * j, 16)] = vrows[e, pl.ds(16 * j, 16)] * w
            return 0
        lax.fori_loop(0, B2, scale, 0)

        pltpu.sync_copy(vrows, acc_sh.at[dstb.at[0]], add=True)

        @pl.when(c == 0)
        def _():
            pltpu.sync_copy(ea_hbm.at[pl.ds(base, B2)], eabuf)

            def escale(e, _):
                easc[e, :] = eabuf[e, :] * wbuf[e]
                return 0
            lax.fori_loop(0, B2, escale, 0)
            pltpu.sync_copy(easc, eacc_sh.at[dstb.at[0]], add=True)
        return 0

    lax.fori_loop(0, NB2, block, 0)

    plsc.subcore_barrier()

    pltpu.sync_copy(acc_sh.at[pl.ds(s * rows_per_tile, rows_per_tile)],
                    out_hbm.at[pl.ds(coff + s * rows_per_tile, rows_per_tile)])

    @pl.when(c == 0)
    def _():
        pltpu.sync_copy(eacc_sh.at[pl.ds(s * rows_per_tile, rows_per_tile)],
                        eacc_hbm.at[pl.ds(s * rows_per_tile, rows_per_tile)])


_sc_pass2 = functools.partial(
    pl.kernel,
    out_type=[
        jax.ShapeDtypeStruct((2 * N, CH), jnp.float32),
        jax.ShapeDtypeStruct((N, EDP), jnp.float32),
    ],
    mesh=_mesh,
    scratch_types=[
        pltpu.VMEM_SHARED((N, CH), jnp.float32),
        pltpu.VMEM_SHARED((N, EDP), jnp.float32),
        pltpu.VMEM((N,), jnp.float32),
        pltpu.VMEM((N,), jnp.float32),
        pltpu.VMEM((125, CH), jnp.float32),
        pltpu.VMEM((N // NS, EDP), jnp.float32),
        pltpu.VMEM((B2,), jnp.int32),
        pltpu.VMEM((1, B2), jnp.int32),
        pltpu.VMEM((B2,), jnp.float32),
        pltpu.VMEM((B2,), jnp.float32),
        pltpu.VMEM((B2, CH), jnp.float32),
        pltpu.VMEM((B2, EDP), jnp.float32),
        pltpu.VMEM((B2, EDP), jnp.float32),
        pltpu.SemaphoreType.DMA,
    ],
)(_sc_pass2_body)


def _mlp_body(x_ref, h0, h1, eacc, we16, wsk, bsk, w1, b1, w2, b2, w3, b3,
              out_ref):
    x = x_ref[...]
    h = jnp.concatenate([h0[...][:, :140], h1[...][:, :140]], axis=1)
    h = h + jnp.dot(eacc[...], we16[...], preferred_element_type=jnp.float32)
    h = h + jnp.dot(x, wsk[...], preferred_element_type=jnp.float32) + bsk[...]
    t = jnp.maximum(jnp.dot(h, w1[...], preferred_element_type=jnp.float32)
                    + b1[...], 0.0)
    t = jnp.maximum(jnp.dot(t, w2[...], preferred_element_type=jnp.float32)
                    + b2[...], 0.0)
    lg = jnp.dot(t, w3[...], preferred_element_type=jnp.float32) + b3[...]
    m = jnp.max(lg, axis=-1, keepdims=True)
    ex = jnp.exp(lg - m)
    out_ref[...] = ex / jnp.sum(ex, axis=-1, keepdims=True)


def _phase_c(x, h0, h1, eacc, We16, Wskip, bskip, W1, b1, W2, b2, W3, b3):
    bn = 1000
    grid = (N // bn,)
    full = lambda shp: pl.BlockSpec(shp, lambda i: (0, 0))
    blk = lambda w: pl.BlockSpec((bn, w), lambda i: (i, 0))
    return pl.pallas_call(
        _mlp_body,
        grid=grid,
        in_specs=[
            blk(D), blk(CH), blk(CH), blk(EDP),
            full((EDP, C)), full((D, C)), full((1, C)),
            full((C, 64)), full((1, 64)),
            full((64, 16)), full((1, 16)),
            full((16, 8)), full((1, 8)),
        ],
        out_specs=blk(8),
        out_shape=jax.ShapeDtypeStruct((N, 8), jnp.float32),
    )(x, h0, h1, eacc, We16, Wskip, bskip.reshape(1, C),
      W1, b1.reshape(1, 64), W2, b2.reshape(1, 16), W3, b3.reshape(1, 8))


@jax.jit
def kernel(x, edge_index, edge_attr, Wq, bq, Wk, bk, Wv, bv, We, Wskip, bskip,
           W1, b1, W2, b2, W3, b3):
    WeT16 = jnp.pad(We.T, ((0, 0), (0, EDP - 6)))
    We16 = jnp.pad(We, ((0, EDP - 6), (0, 0)))
    ea16 = jnp.pad(edge_attr, ((0, 0), (0, EDP - 6)))

    qpad, kpad, qe16, vpad = _phase_a(x, Wq, bq, Wk, bk, Wv, bv, WeT16)
    vab = jnp.concatenate([vpad[:, :CH], vpad[:, CH:]], axis=0)

    p, dparts = _sc_pass1(qpad, kpad, qe16, edge_index, ea16)
    out2, eacc = _sc_pass2(vab, edge_index, ea16, p, dparts)

    h0 = out2[:N]
    h1 = out2[N:]
    return _phase_c(x, h0, h1, eacc, We16, Wskip, bskip, W1, b1, W2, b2, W3,
                    b3)


# speedup vs baseline: 2.1289x; 2.1289x over previous
import functools

import jax
import jax.numpy as jnp
import numpy as np
from jax import lax
from jax.experimental import pallas as pl
from jax.experimental.pallas import tpu as pltpu
from jax.experimental.pallas import tpu_sc as plsc

N = 10000
E = 160000
D = 128
C = 280
CP = 288
CH = 144
EDP = 16
NC = 2
NS = 16
NW = NC * NS
INV_SQRT_C = 1.0 / float(np.sqrt(float(C)))

EPT1 = E // NW
B1 = 40
BR1 = 48
NB1 = EPT1 // B1
EPT2 = E // NS
B2 = 80
NB2 = EPT2 // B2

_mesh = plsc.VectorSubcoreMesh(
    core_axis_name="c", subcore_axis_name="s", num_cores=NC, num_subcores=NS)


def _projc_body(x_ref, wq, bq, wk, bk, wv, bv, wet, q_out, k_out, qe_out, v_out):
    x = x_ref[...]
    z8 = jnp.zeros((x.shape[0], CP - C), jnp.float32)
    z4 = jnp.zeros((x.shape[0], 4), jnp.float32)
    qb = jnp.dot(x, wq[...], preferred_element_type=jnp.float32, precision=lax.Precision.HIGHEST) + bq[...]
    kb = jnp.dot(x, wk[...], preferred_element_type=jnp.float32, precision=lax.Precision.HIGHEST) + bk[...]
    vb = jnp.dot(x, wv[...], preferred_element_type=jnp.float32, precision=lax.Precision.HIGHEST) + bv[...]
    q_out[...] = jnp.concatenate([qb, z8], axis=1)
    k_out[...] = jnp.concatenate([kb, z8], axis=1)
    qe_out[...] = jnp.dot(qb, wet[...], preferred_element_type=jnp.float32, precision=lax.Precision.HIGHEST)
    v_out[...] = jnp.concatenate(
        [vb[:, :140], z4, vb[:, 140:], z4], axis=1)


def _phase_a(x, Wq, bq, Wk, bk, Wv, bv, WeT16):
    bn = 1000
    grid = (N // bn,)
    full = lambda shp: pl.BlockSpec(shp, lambda i: (0, 0))
    return pl.pallas_call(
        _projc_body,
        grid=grid,
        in_specs=[
            pl.BlockSpec((bn, D), lambda i: (i, 0)),
            full((D, C)), full((1, C)),
            full((D, C)), full((1, C)),
            full((D, C)), full((1, C)),
            full((C, EDP)),
        ],
        out_specs=[
            pl.BlockSpec((bn, CP), lambda i: (i, 0)),
            pl.BlockSpec((bn, CP), lambda i: (i, 0)),
            pl.BlockSpec((bn, EDP), lambda i: (i, 0)),
            pl.BlockSpec((bn, CP), lambda i: (i, 0)),
        ],
        out_shape=[
            jax.ShapeDtypeStruct((N, CP), jnp.float32),
            jax.ShapeDtypeStruct((N, CP), jnp.float32),
            jax.ShapeDtypeStruct((N, EDP), jnp.float32),
            jax.ShapeDtypeStruct((N, CP), jnp.float32),
        ],
    )(x, Wq, bq.reshape(1, C), Wk, bk.reshape(1, C), Wv, bv.reshape(1, C),
      WeT16)


def _sc_pass1_body(q_hbm, k_hbm, qe_hbm, src_hbm, dst_hbm, ea_hbm,
                   p_hbm, dparts_hbm,
                   qbuf, kbuf, qebuf, eabuf, srcb, dstb, matbuf, pbuf, denomb,
                   sem):
    wid = lax.axis_index("c") * NS + lax.axis_index("s")
    z16 = jnp.zeros((16,), jnp.float32)
    zi16 = jnp.zeros((16,), jnp.int32)
    idx0 = lax.iota(jnp.int32, 16) * 16
    tailmask = lax.iota(jnp.int32, 16) < (B1 - 2 * 16)

    def zdenom(i, _):
        denomb[pl.ds(i * 16, 16)] = z16
        return 0
    lax.fori_loop(0, N // 16, zdenom, 0)

    for e in range(B1, BR1):
        for j in range(CP // 16):
            kbuf[e, pl.ds(16 * j, 16)] = z16
            qbuf[e, pl.ds(16 * j, 16)] = z16
        qebuf[e, :] = z16
        eabuf[e, :] = z16
    dstb[pl.ds(32, 16)] = zi16

    def block(i, _):
        base = wid * EPT1 + i * B1
        pltpu.sync_copy(src_hbm.at[pl.ds(base, B1)], srcb)
        pltpu.sync_copy(dst_hbm.at[pl.ds(base, B1)], dstb.at[pl.ds(0, B1)])
        pltpu.sync_copy(ea_hbm.at[pl.ds(base, B1)], eabuf.at[pl.ds(0, B1)])
        c1 = pltpu.async_copy(k_hbm.at[srcb], kbuf.at[pl.ds(0, B1)], sem)
        c2 = pltpu.async_copy(q_hbm.at[dstb.at[pl.ds(0, B1)]],
                              qbuf.at[pl.ds(0, B1)], sem)
        c3 = pltpu.async_copy(qe_hbm.at[dstb.at[pl.ds(0, B1)]],
                              qebuf.at[pl.ds(0, B1)], sem)
        c1.wait()
        c2.wait()
        c3.wait()

        def group(g, _):
            for e16 in range(16):
                e = g * 16 + e16
                acc = kbuf[e, pl.ds(0, 16)] * qbuf[e, pl.ds(0, 16)]
                for j in range(1, CP // 16):
                    acc = acc + (kbuf[e, pl.ds(16 * j, 16)]
                                 * qbuf[e, pl.ds(16 * j, 16)])
                acc = acc + qebuf[e, :] * eabuf[e, :]
                matbuf[pl.ds(e16 * 16, 16)] = acc
            s16 = plsc.load_gather(matbuf, [idx0])
            for l in range(1, 16):
                s16 = s16 + plsc.load_gather(matbuf, [idx0 + l])
            p16 = jnp.exp(s16 * INV_SQRT_C)
            pbuf[pl.ds(g * 16, 16)] = p16
            return p16

        for g in range(2):
            group(g, 0)
            plsc.addupdate_scatter(denomb, [dstb[pl.ds(g * 16, 16)]],
                                   pbuf[pl.ds(g * 16, 16)])
        ptail = group(2, 0)
        plsc.addupdate_scatter(denomb, [dstb[pl.ds(32, 16)]], ptail,
                               mask=tailmask)

        pltpu.sync_copy(pbuf.at[pl.ds(0, B1)], p_hbm.at[pl.ds(base, B1)])
        return 0

    lax.fori_loop(0, NB1, block, 0)
    pltpu.sync_copy(denomb, dparts_hbm.at[wid])


_sc_pass1 = functools.partial(
    pl.kernel,
    out_type=[
        jax.ShapeDtypeStruct((E,), jnp.float32),
        jax.ShapeDtypeStruct((NW, N), jnp.float32),
    ],
    mesh=_mesh,
    compiler_params=pltpu.CompilerParams(needs_layout_passes=False, use_tc_tiling_on_sc=False),
    scratch_types=[
        pltpu.VMEM((BR1, CP), jnp.float32),
        pltpu.VMEM((BR1, CP), jnp.float32),
        pltpu.VMEM((BR1, EDP), jnp.float32),
        pltpu.VMEM((BR1, EDP), jnp.float32),
        pltpu.VMEM((B1,), jnp.int32),
        pltpu.VMEM((BR1,), jnp.int32),
        pltpu.VMEM((256,), jnp.float32),
        pltpu.VMEM((BR1,), jnp.float32),
        pltpu.VMEM((N,), jnp.float32),
        pltpu.SemaphoreType.DMA,
    ],
)(_sc_pass1_body)


def _sc_pass2_body(vab_hbm, src_hbm, dst_hbm, ea_hbm, p_hbm, dparts_hbm,
                   out_hbm, eacc_hbm,
                   acc_sh, eacc_sh,
                   dbuf, tmp, srcb, dstb, pbuf, wbuf,
                   vrows, eabuf, easc, sem):
    c = lax.axis_index("c")
    s = lax.axis_index("s")
    z16 = jnp.zeros((16,), jnp.float32)
    rows_per_tile = N // NS

    def zrow(r, _):
        for j in range(CH // 16):
            vrows[r, pl.ds(16 * j, 16)] = z16
        easc[r, :] = z16
        return 0
    lax.fori_loop(0, B2, zrow, 0)

    for t in range(7):
        pltpu.sync_copy(vrows, acc_sh.at[pl.ds(s * rows_per_tile + t * B2, B2)])
    pltpu.sync_copy(vrows.at[pl.ds(0, rows_per_tile - 7 * B2)],
                    acc_sh.at[pl.ds(s * rows_per_tile + 7 * B2,
                                    rows_per_tile - 7 * B2)])

    @pl.when(c == 0)
    def _():
        for t in range(7):
            pltpu.sync_copy(easc,
                            eacc_sh.at[pl.ds(s * rows_per_tile + t * B2, B2)])
        pltpu.sync_copy(easc.at[pl.ds(0, rows_per_tile - 7 * B2)],
                        eacc_sh.at[pl.ds(s * rows_per_tile + 7 * B2,
                                         rows_per_tile - 7 * B2)])

    def zd(i, _):
        dbuf[pl.ds(i * 16, 16)] = z16
        return 0
    lax.fori_loop(0, N // 16, zd, 0)

    def addpart(r, _):
        def chunk(k, _):
            pltpu.sync_copy(dparts_hbm.at[r, pl.ds(k * 2000, 2000)], tmp)

            def addv(j, _):
                o = k * 2000 + j * 16
                dbuf[pl.ds(o, 16)] = dbuf[pl.ds(o, 16)] + tmp[pl.ds(j * 16, 16)]
                return 0
            lax.fori_loop(0, 125, addv, 0)
            return 0
        lax.fori_loop(0, 5, chunk, 0)
        return 0
    lax.fori_loop(0, NW, addpart, 0)

    plsc.subcore_barrier()

    coff = c * N

    def block(i, _):
        base = s * EPT2 + i * B2
        pltpu.sync_copy(src_hbm.at[pl.ds(base, B2)], srcb)
        pltpu.sync_copy(dst_hbm.at[pl.ds(base, B2)], dstb.at[0])
        pltpu.sync_copy(p_hbm.at[pl.ds(base, B2)], pbuf)

        def soff(j, _):
            srcb[pl.ds(16 * j, 16)] = srcb[pl.ds(16 * j, 16)] + coff
            return 0
        lax.fori_loop(0, B2 // 16, soff, 0)

        pltpu.async_copy(vab_hbm.at[srcb], vrows, sem).wait()

        def wvec(j, _):
            d16 = plsc.load_gather(dbuf, [dstb[0, pl.ds(16 * j, 16)]])
            wbuf[pl.ds(16 * j, 16)] = pbuf[pl.ds(16 * j, 16)] / (d16 + 1e-16)
            return 0
        lax.fori_loop(0, B2 // 16, wvec, 0)

        def scale(g, _):
            w16 = wbuf[pl.ds(g * 16, 16)]
            for e16 in range(16):
                e = g * 16 + e16
                w = w16[e16]
                for j in range(CH // 16):
                    vrows[e, pl.ds(16 * j, 16)] = vrows[e, pl.ds(16 * j, 16)] * w
            return 0
        lax.fori_loop(0, B2 // 16, scale, 0)

        pltpu.sync_copy(vrows, acc_sh.at[dstb.at[0]], add=True)

        @pl.when(c == 0)
        def _():
            pltpu.sync_copy(ea_hbm.at[pl.ds(base, B2)], eabuf)

            def escale(g, _):
                w16 = wbuf[pl.ds(g * 16, 16)]
                for e16 in range(16):
                    e = g * 16 + e16
                    easc[e, :] = eabuf[e, :] * w16[e16]
                return 0
            lax.fori_loop(0, B2 // 16, escale, 0)
            pltpu.sync_copy(easc, eacc_sh.at[dstb.at[0]], add=True)
        return 0

    lax.fori_loop(0, NB2, block, 0)

    plsc.subcore_barrier()

    pltpu.sync_copy(acc_sh.at[pl.ds(s * rows_per_tile, rows_per_tile)],
                    out_hbm.at[pl.ds(coff + s * rows_per_tile, rows_per_tile)])

    @pl.when(c == 0)
    def _():
        pltpu.sync_copy(eacc_sh.at[pl.ds(s * rows_per_tile, rows_per_tile)],
                        eacc_hbm.at[pl.ds(s * rows_per_tile, rows_per_tile)])


_sc_pass2 = functools.partial(
    pl.kernel,
    out_type=[
        jax.ShapeDtypeStruct((2 * N, CH), jnp.float32),
        jax.ShapeDtypeStruct((N, EDP), jnp.float32),
    ],
    mesh=_mesh,
    compiler_params=pltpu.CompilerParams(needs_layout_passes=False, use_tc_tiling_on_sc=False),
    scratch_types=[
        pltpu.VMEM_SHARED((N, CH), jnp.float32),
        pltpu.VMEM_SHARED((N, EDP), jnp.float32),
        pltpu.VMEM((N,), jnp.float32),
        pltpu.VMEM((2000,), jnp.float32),
        pltpu.VMEM((B2,), jnp.int32),
        pltpu.VMEM((1, B2), jnp.int32),
        pltpu.VMEM((B2,), jnp.float32),
        pltpu.VMEM((B2,), jnp.float32),
        pltpu.VMEM((B2, CH), jnp.float32),
        pltpu.VMEM((B2, EDP), jnp.float32),
        pltpu.VMEM((B2, EDP), jnp.float32),
        pltpu.SemaphoreType.DMA,
    ],
)(_sc_pass2_body)


def _mlp_body(x_ref, h0, h1, eacc, we16, wsk, bsk, w1, b1, w2, b2, w3, b3,
              out_ref):
    x = x_ref[...]
    h = jnp.concatenate([h0[...][:, :140], h1[...][:, :140]], axis=1)
    h = h + jnp.dot(eacc[...], we16[...], preferred_element_type=jnp.float32, precision=lax.Precision.HIGHEST)
    h = h + jnp.dot(x, wsk[...], preferred_element_type=jnp.float32, precision=lax.Precision.HIGHEST) + bsk[...]
    t = jnp.maximum(jnp.dot(h, w1[...], preferred_element_type=jnp.float32, precision=lax.Precision.HIGHEST)
                    + b1[...], 0.0)
    t = jnp.maximum(jnp.dot(t, w2[...], preferred_element_type=jnp.float32, precision=lax.Precision.HIGHEST)
                    + b2[...], 0.0)
    lg = jnp.dot(t, w3[...], preferred_element_type=jnp.float32, precision=lax.Precision.HIGHEST) + b3[...]
    m = jnp.max(lg, axis=-1, keepdims=True)
    ex = jnp.exp(lg - m)
    out_ref[...] = ex / jnp.sum(ex, axis=-1, keepdims=True)


def _phase_c(x, h0, h1, eacc, We16, Wskip, bskip, W1, b1, W2, b2, W3, b3):
    bn = 1000
    grid = (N // bn,)
    full = lambda shp: pl.BlockSpec(shp, lambda i: (0, 0))
    blk = lambda w: pl.BlockSpec((bn, w), lambda i: (i, 0))
    return pl.pallas_call(
        _mlp_body,
        grid=grid,
        in_specs=[
            blk(D), blk(CH), blk(CH), blk(EDP),
            full((EDP, C)), full((D, C)), full((1, C)),
            full((C, 64)), full((1, 64)),
            full((64, 16)), full((1, 16)),
            full((16, 8)), full((1, 8)),
        ],
        out_specs=blk(8),
        out_shape=jax.ShapeDtypeStruct((N, 8), jnp.float32),
    )(x, h0, h1, eacc, We16, Wskip, bskip.reshape(1, C),
      W1, b1.reshape(1, 64), W2, b2.reshape(1, 16), W3, b3.reshape(1, 8))


@jax.jit
def kernel(x, edge_index, edge_attr, Wq, bq, Wk, bk, Wv, bv, We, Wskip, bskip,
           W1, b1, W2, b2, W3, b3):
    WeT16 = jnp.pad(We.T, ((0, 0), (0, EDP - 6)))
    We16 = jnp.pad(We, ((0, EDP - 6), (0, 0)))
    ea16 = jnp.pad(edge_attr, ((0, 0), (0, EDP - 6)))

    qpad, kpad, qe16, vpad = _phase_a(x, Wq, bq, Wk, bk, Wv, bv, WeT16)
    vab = jnp.concatenate([vpad[:, :CH], vpad[:, CH:]], axis=0)

    src = edge_index[0]
    dst = edge_index[1]
    p, dparts = _sc_pass1(qpad, kpad, qe16, src, dst, ea16)
    out2, eacc = _sc_pass2(vab, src, dst, ea16, p, dparts)

    h0 = out2[:N]
    h1 = out2[N:]
    return _phase_c(x, h0, h1, eacc, We16, Wskip, bskip, W1, b1, W2, b2, W3,
                    b3)


# speedup vs baseline: 3.2374x; 1.5207x over previous
import functools

import jax
import jax.numpy as jnp
import numpy as np
from jax import lax
from jax.experimental import pallas as pl
from jax.experimental.pallas import tpu as pltpu
from jax.experimental.pallas import tpu_sc as plsc

N = 10000
E = 160000
D = 128
C = 280
CP = 288
CH = 144
GW = 144
EDP = 16
NC = 2
NS = 16
NW = NC * NS
INV_SQRT_C = 1.0 / float(np.sqrt(float(C)))

EPT1 = E // NW
B1 = 40
BR1 = 48
NB1 = EPT1 // B1
EPT2 = E // NS
B2 = 80
NB2 = EPT2 // B2

_mesh = plsc.VectorSubcoreMesh(
    core_axis_name="c", subcore_axis_name="s", num_cores=NC, num_subcores=NS)
_sc_params = pltpu.CompilerParams(
    needs_layout_passes=False, use_tc_tiling_on_sc=False)


def _m_body(wq1, wk1, m_out):
    m_out[...] = lax.dot_general(
        wq1[...], wk1[...], (((1,), (1,)), ((), ())),
        preferred_element_type=jnp.float32)


def _phase_m(Wq1, Wk1):
    full = lambda shp: pl.BlockSpec(shp, lambda: (0, 0))
    return pl.pallas_call(
        _m_body,
        in_specs=[full((136, C)), full((136, C))],
        out_specs=full((136, 136)),
        out_shape=jax.ShapeDtypeStruct((136, 136), jnp.float32),
    )(Wq1, Wk1)


def _projc_body(x_ref, m_ref, wq, bq, wet, wv, bv, gx_out, v_out):
    x = x_ref[...]
    m = m_ref[...]
    g = jnp.dot(x, m[:128, :], preferred_element_type=jnp.float32) + m[128:129, :]
    qb = jnp.dot(x, wq[...], preferred_element_type=jnp.float32) + bq[...]
    qe = jnp.dot(qb, wet[...], preferred_element_type=jnp.float32)
    z9 = jnp.zeros((x.shape[0], 9), jnp.float32)
    z4 = jnp.zeros((x.shape[0], 4), jnp.float32)
    gx_out[...] = jnp.concatenate(
        [g[:, :128], qe[:, :6], g[:, 128:129], z9], axis=1)
    vb = jnp.dot(x, wv[...], preferred_element_type=jnp.float32) + bv[...]
    v_out[...] = jnp.concatenate(
        [vb[:, :140], z4, vb[:, 140:], z4], axis=1)


def _phase_a(x, M, Wq, bq, WeT16, Wv, bv):
    bn = 1000
    grid = (N // bn,)
    full = lambda shp: pl.BlockSpec(shp, lambda i: (0, 0))
    return pl.pallas_call(
        _projc_body,
        grid=grid,
        in_specs=[
            pl.BlockSpec((bn, D), lambda i: (i, 0)),
            full((136, 136)),
            full((D, C)), full((1, C)),
            full((C, EDP)),
            full((D, C)), full((1, C)),
        ],
        out_specs=[
            pl.BlockSpec((bn, GW), lambda i: (i, 0)),
            pl.BlockSpec((bn, CP), lambda i: (i, 0)),
        ],
        out_shape=[
            jax.ShapeDtypeStruct((N, GW), jnp.float32),
            jax.ShapeDtypeStruct((N, CP), jnp.float32),
        ],
    )(x, M, Wq, bq.reshape(1, C), WeT16, Wv, bv.reshape(1, C))


def _sc_pass1_body(gx_hbm, x_hbm, src_hbm, dst_hbm, ea_hbm,
                   p_hbm, dparts_hbm,
                   xk0, xk1, gx0, gx1, ea0, ea1, src0, src1, dst0, dst1,
                   matbuf, pbuf, denomb, sem0, sem1):
    xkb, gxb, eab, srcb, dstb = [xk0, xk1], [gx0, gx1], [ea0, ea1], \
        [src0, src1], [dst0, dst1]
    sems = [sem0, sem1]
    wid = lax.axis_index("c") * NS + lax.axis_index("s")
    z16 = jnp.zeros((16,), jnp.float32)
    zi16 = jnp.zeros((16,), jnp.int32)
    idx0 = lax.iota(jnp.int32, 16) * 16
    tailmask = lax.iota(jnp.int32, 16) < (B1 - 2 * 16)

    def zdenom(i, _):
        denomb[pl.ds(i * 16, 16)] = z16
        return 0
    lax.fori_loop(0, N // 16, zdenom, 0)

    for sl in range(2):
        for e in range(B1, BR1):
            for j in range(D // 16):
                xkb[sl][e, pl.ds(16 * j, 16)] = z16
            for j in range(GW // 16):
                gxb[sl][e, pl.ds(16 * j, 16)] = z16
            eab[sl][e, :] = z16
        dstb[sl][pl.ds(32, 16)] = zi16

    def fire(i, sl):
        base = wid * EPT1 + i * B1
        pltpu.sync_copy(src_hbm.at[pl.ds(base, B1)], srcb[sl])
        pltpu.sync_copy(dst_hbm.at[pl.ds(base, B1)], dstb[sl].at[pl.ds(0, B1)])
        pltpu.sync_copy(ea_hbm.at[pl.ds(base, B1)], eab[sl].at[pl.ds(0, B1)])
        pltpu.async_copy(x_hbm.at[srcb[sl]], xkb[sl].at[pl.ds(0, B1)], sems[sl])
        pltpu.async_copy(gx_hbm.at[dstb[sl].at[pl.ds(0, B1)]],
                         gxb[sl].at[pl.ds(0, B1)], sems[sl])

    def wait(sl):
        pltpu.make_async_copy(x_hbm.at[srcb[sl]], xkb[sl].at[pl.ds(0, B1)],
                              sems[sl]).wait()
        pltpu.make_async_copy(gx_hbm.at[dstb[sl].at[pl.ds(0, B1)]],
                              gxb[sl].at[pl.ds(0, B1)], sems[sl]).wait()

    def compute(i, sl):
        base = wid * EPT1 + i * B1
        xk, gx, ea, dst = xkb[sl], gxb[sl], eab[sl], dstb[sl]

        def group(g):
            for e16 in range(16):
                e = g * 16 + e16
                acc = gx[e, pl.ds(0, 16)] * xk[e, pl.ds(0, 16)]
                for j in range(1, D // 16):
                    acc = acc + (gx[e, pl.ds(16 * j, 16)]
                                 * xk[e, pl.ds(16 * j, 16)])
                acc = acc + gx[e, pl.ds(128, 16)] * ea[e, :]
                matbuf[pl.ds(e16 * 16, 16)] = acc
            s16 = plsc.load_gather(matbuf, [idx0])
            for l in range(1, 16):
                s16 = s16 + plsc.load_gather(matbuf, [idx0 + l])
            p16 = jnp.exp(s16 * INV_SQRT_C)
            pbuf[pl.ds(g * 16, 16)] = p16
            return p16

        for g in range(2):
            group(g)
            plsc.addupdate_scatter(denomb, [dst[pl.ds(g * 16, 16)]],
                                   pbuf[pl.ds(g * 16, 16)])
        ptail = group(2)
        plsc.addupdate_scatter(denomb, [dst[pl.ds(32, 16)]], ptail,
                               mask=tailmask)
        pltpu.sync_copy(pbuf.at[pl.ds(0, B1)], p_hbm.at[pl.ds(base, B1)])

    fire(0, 0)

    def pair(i2, _):
        i = i2 * 2
        fire(i + 1, 1)
        wait(0)
        compute(i, 0)
        fire(i + 2, 0)
        wait(1)
        compute(i + 1, 1)
        return 0
    lax.fori_loop(0, NB1 // 2, pair, 0)
    wait(0)
    compute(NB1 - 1, 0)

    pltpu.sync_copy(denomb, dparts_hbm.at[wid])


_sc_pass1 = functools.partial(
    pl.kernel,
    out_type=[
        jax.ShapeDtypeStruct((E,), jnp.float32),
        jax.ShapeDtypeStruct((NW, N), jnp.float32),
    ],
    mesh=_mesh,
    compiler_params=_sc_params,
    scratch_types=[
        pltpu.VMEM((BR1, D), jnp.float32),
        pltpu.VMEM((BR1, D), jnp.float32),
        pltpu.VMEM((BR1, GW), jnp.float32),
        pltpu.VMEM((BR1, GW), jnp.float32),
        pltpu.VMEM((BR1, EDP), jnp.float32),
        pltpu.VMEM((BR1, EDP), jnp.float32),
        pltpu.VMEM((B1,), jnp.int32),
        pltpu.VMEM((B1,), jnp.int32),
        pltpu.VMEM((BR1,), jnp.int32),
        pltpu.VMEM((BR1,), jnp.int32),
        pltpu.VMEM((256,), jnp.float32),
        pltpu.VMEM((BR1,), jnp.float32),
        pltpu.VMEM((N,), jnp.float32),
        pltpu.SemaphoreType.DMA,
        pltpu.SemaphoreType.DMA,
    ],
)(_sc_pass1_body)


def _sc_pass2_body(vab_hbm, src_hbm, dst_hbm, ea_hbm, p_hbm,
                   out_hbm, eacc_hbm,
                   acc_sh, eacc_sh,
                   vr0, vr1, src0, src1, dst0, dst1, pb0, pb1, ea0, ea1,
                   easc, sem0, sem1):
    vrows, srcb, dstb, pbuf, eab = [vr0, vr1], [src0, src1], [dst0, dst1], \
        [pb0, pb1], [ea0, ea1]
    sems = [sem0, sem1]
    c = lax.axis_index("c")
    s = lax.axis_index("s")
    z16 = jnp.zeros((16,), jnp.float32)
    rows_per_tile = N // NS
    coff = c * N

    def zrow(r, _):
        for j in range(CH // 16):
            vr0[r, pl.ds(16 * j, 16)] = z16
        easc[r, :] = z16
        return 0
    lax.fori_loop(0, B2, zrow, 0)

    for t in range(7):
        pltpu.sync_copy(vr0, acc_sh.at[pl.ds(s * rows_per_tile + t * B2, B2)])
    pltpu.sync_copy(vr0.at[pl.ds(0, rows_per_tile - 7 * B2)],
                    acc_sh.at[pl.ds(s * rows_per_tile + 7 * B2,
                                    rows_per_tile - 7 * B2)])

    @pl.when(c == 0)
    def _():
        for t in range(7):
            pltpu.sync_copy(easc,
                            eacc_sh.at[pl.ds(s * rows_per_tile + t * B2, B2)])
        pltpu.sync_copy(easc.at[pl.ds(0, rows_per_tile - 7 * B2)],
                        eacc_sh.at[pl.ds(s * rows_per_tile + 7 * B2,
                                         rows_per_tile - 7 * B2)])

    plsc.subcore_barrier()

    def fire(i, sl):
        base = s * EPT2 + i * B2
        pltpu.sync_copy(src_hbm.at[pl.ds(base, B2)], srcb[sl])
        pltpu.sync_copy(dst_hbm.at[pl.ds(base, B2)], dstb[sl].at[0])
        pltpu.sync_copy(p_hbm.at[pl.ds(base, B2)], pbuf[sl])
        pltpu.sync_copy(ea_hbm.at[pl.ds(base, B2)], eab[sl])

        def soff(j, _):
            srcb[sl][pl.ds(16 * j, 16)] = srcb[sl][pl.ds(16 * j, 16)] + coff
            return 0
        lax.fori_loop(0, B2 // 16, soff, 0)
        pltpu.async_copy(vab_hbm.at[srcb[sl]], vrows[sl], sems[sl])

    def wait(sl):
        pltpu.make_async_copy(vab_hbm.at[srcb[sl]], vrows[sl],
                              sems[sl]).wait()

    def compute(i, sl):
        vr, dst, pb, ea = vrows[sl], dstb[sl], pbuf[sl], eab[sl]

        def scale(g, _):
            w16 = pb[pl.ds(g * 16, 16)]
            for e16 in range(16):
                e = g * 16 + e16
                w = w16[e16]
                for j in range(CH // 16):
                    vr[e, pl.ds(16 * j, 16)] = vr[e, pl.ds(16 * j, 16)] * w
            return 0
        lax.fori_loop(0, B2 // 16, scale, 0)
        pltpu.sync_copy(vr, acc_sh.at[dst.at[0]], add=True)

        @pl.when(c == 0)
        def _():
            def escale(g, _):
                w16 = pb[pl.ds(g * 16, 16)]
                for e16 in range(16):
                    e = g * 16 + e16
                    easc[e, :] = ea[e, :] * w16[e16]
                return 0
            lax.fori_loop(0, B2 // 16, escale, 0)
            pltpu.sync_copy(easc, eacc_sh.at[dst.at[0]], add=True)

    fire(0, 0)

    def pair(i2, _):
        i = i2 * 2
        fire(i + 1, 1)
        wait(0)
        compute(i, 0)
        fire(i + 2, 0)
        wait(1)
        compute(i + 1, 1)
        return 0
    lax.fori_loop(0, NB2 // 2, pair, 0)
    wait(0)
    compute(NB2 - 1, 0)

    plsc.subcore_barrier()

    pltpu.sync_copy(acc_sh.at[pl.ds(s * rows_per_tile, rows_per_tile)],
                    out_hbm.at[pl.ds(coff + s * rows_per_tile, rows_per_tile)])

    @pl.when(c == 0)
    def _():
        pltpu.sync_copy(eacc_sh.at[pl.ds(s * rows_per_tile, rows_per_tile)],
                        eacc_hbm.at[pl.ds(s * rows_per_tile, rows_per_tile)])


_sc_pass2 = functools.partial(
    pl.kernel,
    out_type=[
        jax.ShapeDtypeStruct((2 * N, CH), jnp.float32),
        jax.ShapeDtypeStruct((N, EDP), jnp.float32),
    ],
    mesh=_mesh,
    compiler_params=_sc_params,
    scratch_types=[
        pltpu.VMEM_SHARED((N, CH), jnp.float32),
        pltpu.VMEM_SHARED((N, EDP), jnp.float32),
        pltpu.VMEM((B2, CH), jnp.float32),
        pltpu.VMEM((B2, CH), jnp.float32),
        pltpu.VMEM((B2,), jnp.int32),
        pltpu.VMEM((B2,), jnp.int32),
        pltpu.VMEM((1, B2), jnp.int32),
        pltpu.VMEM((1, B2), jnp.int32),
        pltpu.VMEM((B2,), jnp.float32),
        pltpu.VMEM((B2,), jnp.float32),
        pltpu.VMEM((B2, EDP), jnp.float32),
        pltpu.VMEM((B2, EDP), jnp.float32),
        pltpu.VMEM((B2, EDP), jnp.float32),
        pltpu.SemaphoreType.DMA,
        pltpu.SemaphoreType.DMA,
    ],
)(_sc_pass2_body)


def _mlp_body(x_ref, h0, h1, eacc, dparts, we16, wsk, bsk, w1, b1, w2, b2,
              w3, b3, out_ref):
    x = x_ref[...]
    dn = jnp.sum(dparts[0], axis=1)
    inv = 1.0 / (dn + 1e-16)
    h = jnp.concatenate([h0[...][:, :140], h1[...][:, :140]], axis=1)
    h = h + jnp.dot(eacc[...], we16[...], preferred_element_type=jnp.float32)
    h = h * inv[:, None]
    h = h + jnp.dot(x, wsk[...], preferred_element_type=jnp.float32) + bsk[...]
    t = jnp.maximum(jnp.dot(h, w1[...], preferred_element_type=jnp.float32)
                    + b1[...], 0.0)
    t = jnp.maximum(jnp.dot(t, w2[...], preferred_element_type=jnp.float32)
                    + b2[...], 0.0)
    lg = jnp.dot(t, w3[...], preferred_element_type=jnp.float32) + b3[...]
    m = jnp.max(lg, axis=-1, keepdims=True)
    ex = jnp.exp(lg - m)
    out_ref[...] = ex / jnp.sum(ex, axis=-1, keepdims=True)


def _phase_c(x, h0, h1, eacc, dparts, We16, Wskip, bskip, W1, b1, W2, b2,
             W3, b3):
    bn = 1000
    grid = (N // bn,)
    full = lambda shp: pl.BlockSpec(shp, lambda i: (0, 0))
    blk = lambda w: pl.BlockSpec((bn, w), lambda i: (i, 0))
    return pl.pallas_call(
        _mlp_body,
        grid=grid,
        in_specs=[
            blk(D), blk(CH), blk(CH), blk(EDP),
            pl.BlockSpec((1, bn, NW), lambda i: (i, 0, 0)),
            full((EDP, C)), full((D, C)), full((1, C)),
            full((C, 64)), full((1, 64)),
            full((64, 16)), full((1, 16)),
            full((16, 8)), full((1, 8)),
        ],
        out_specs=blk(8),
        out_shape=jax.ShapeDtypeStruct((N, 8), jnp.float32),
    )(x, h0, h1, eacc, dparts, We16, Wskip, bskip.reshape(1, C),
      W1, b1.reshape(1, 64), W2, b2.reshape(1, 16), W3, b3.reshape(1, 8))


@jax.jit
def kernel(x, edge_index, edge_attr, Wq, bq, Wk, bk, Wv, bv, We, Wskip, bskip,
           W1, b1, W2, b2, W3, b3):
    WeT16 = jnp.pad(We.T, ((0, 0), (0, EDP - 6)))
    We16 = jnp.pad(We, ((0, EDP - 6), (0, 0)))
    ea16 = jnp.pad(edge_attr, ((0, 0), (0, EDP - 6)))
    ea16 = ea16.at[:, 6].set(1.0)
    Wq1 = jnp.pad(jnp.concatenate([Wq, bq[None, :]], axis=0), ((0, 7), (0, 0)))
    Wk1 = jnp.pad(jnp.concatenate([Wk, bk[None, :]], axis=0), ((0, 7), (0, 0)))

    M = _phase_m(Wq1, Wk1)
    gx, vpad = _phase_a(x, M, Wq, bq, WeT16, Wv, bv)
    vab = jnp.concatenate([vpad[:, :CH], vpad[:, CH:]], axis=0)

    src = edge_index[0]
    dst = edge_index[1]
    p, dparts = _sc_pass1(gx, x, src, dst, ea16)
    out2, eacc = _sc_pass2(vab, src, dst, ea16, p)

    h0 = out2[:N]
    h1 = out2[N:]
    dpt = dparts.T.reshape(N // 1000, 1000, NW)
    return _phase_c(x, h0, h1, eacc, dpt, We16, Wskip, bskip, W1, b1, W2,
                    b2, W3, b3)
